# baseline (speedup 1.0000x reference)
import functools

import jax
import jax.numpy as jnp
from jax import lax
from jax.experimental import pallas as pl
from jax.experimental.pallas import tpu as pltpu
from jax.experimental.pallas import tpu_sc as plsc

N_NODES = 10000
D = 128
N_P = 10240
NC, NS = 2, 16
NW = NC * NS
ROWS_PER_TILE = N_P // NS
CHUNK = 128
CHUNKS = 80
EDGES_P = NW * CHUNKS * CHUNK
ROW_BLK = 1024
GRID = N_P // ROW_BLK

def _sc_mesh():
    return plsc.VectorSubcoreMesh(
        core_axis_name="c", subcore_axis_name="s", num_cores=NC, num_subcores=NS)


def _deg_body(dst_hbm, zeros1_hbm, out_hbm, didx, ones_v, hist):
    cid = lax.axis_index("c")
    sid = lax.axis_index("s")
    wid = sid * NC + cid
    row0 = sid * ROWS_PER_TILE
    pltpu.sync_copy(zeros1_hbm.at[pl.ds(row0, ROWS_PER_TILE)],
                    hist.at[pl.ds(row0, ROWS_PER_TILE)])
    pltpu.sync_copy(dst_hbm.at[wid], didx)
    for i in range(CHUNK // 16):
        ones_v[pl.ds(i * 16, 16)] = jnp.ones((16,), jnp.float32)
    plsc.subcore_barrier()

    def body(j, carry):
        pltpu.sync_copy(ones_v, hist.at[didx.at[j]], add=True)
        return carry

    lax.fori_loop(0, CHUNKS, body, 0)
    plsc.subcore_barrier()
    pltpu.sync_copy(hist.at[pl.ds(row0, ROWS_PER_TILE)],
                    out_hbm.at[cid].at[pl.ds(row0, ROWS_PER_TILE)])


@functools.cache
def _deg_call():
    return pl.kernel(
        _deg_body,
        out_type=jax.ShapeDtypeStruct((NC, N_P), jnp.float32),
        mesh=_sc_mesh(),
        scratch_types=[
            pltpu.VMEM((CHUNKS, CHUNK), jnp.int32),
            pltpu.VMEM((CHUNK,), jnp.float32),
            pltpu.VMEM_SHARED((N_P,), jnp.float32),
        ],
    )


_DST_BITS = 14
_DST_MASK = (1 << _DST_BITS) - 1


def _agg_body(m_hbm, packed_hbm, zeros2_hbm, out_hbm,
              pv, srcb, dstb, gbuf0, gbuf1, acc,
              gsem0, gsem1, ssem0, ssem1):
    cid = lax.axis_index("c")
    sid = lax.axis_index("s")
    wid = sid * NC + cid
    row0 = sid * ROWS_PER_TILE
    pltpu.sync_copy(zeros2_hbm.at[pl.ds(row0, ROWS_PER_TILE)],
                    acc.at[pl.ds(row0, ROWS_PER_TILE)])
    pltpu.sync_copy(packed_hbm.at[wid], pv)

    gbufs = (gbuf0, gbuf1)
    gsems = (gsem0, gsem1)
    ssems = (ssem0, ssem1)

    def unpack(j, slot):
        for k in range(CHUNK // 16):
            w = pv[j, pl.ds(k * 16, 16)]
            srcb[slot, pl.ds(k * 16, 16)] = lax.shift_right_logical(
                w, _DST_BITS)
            dstb[slot, pl.ds(k * 16, 16)] = lax.bitwise_and(w, _DST_MASK)

    def wait_gather(s):
        pltpu.make_async_copy(m_hbm.at[srcb.at[0]], gbufs[s], gsems[s]).wait()

    def wait_scatter(s):
        pltpu.make_async_copy(gbufs[s], acc.at[dstb.at[0]], ssems[s]).wait()

    def half(j, s, i4, first):
        sp = 1 - s
        wait_gather(s)
        pltpu.async_copy(gbufs[s], acc.at[dstb.at[i4]], ssems[s], add=True)
        j1 = jnp.where(j + 1 < CHUNKS, j + 1, 0)
        unpack(j1, (i4 + 1) % 4)
        if not first:
            wait_scatter(sp)
        pltpu.async_copy(m_hbm.at[srcb.at[(i4 + 1) % 4]], gbufs[sp], gsems[sp])

    plsc.subcore_barrier()
    unpack(0, 0)
    pltpu.async_copy(m_hbm.at[srcb.at[0]], gbuf0, gsem0)
    half(0, 0, 0, True)
    half(1, 1, 1, False)

    def body(jj, carry):
        j0 = 2 * jj
        half(j0, 0, j0 % 4, False)
        half(j0 + 1, 1, (j0 + 1) % 4, False)
        return carry

    lax.fori_loop(1, CHUNKS // 2, body, 0)
    wait_gather(0)
    wait_scatter(1)
    plsc.subcore_barrier()
    pltpu.sync_copy(acc.at[pl.ds(row0, ROWS_PER_TILE)],
                    out_hbm.at[cid].at[pl.ds(row0, ROWS_PER_TILE)])


@functools.cache
def _agg_call():
    return pl.kernel(
        _agg_body,
        out_type=jax.ShapeDtypeStruct((NC, N_P, D), jnp.float32),
        mesh=_sc_mesh(),
        scratch_types=[
            pltpu.VMEM((CHUNKS, CHUNK), jnp.int32),
            pltpu.VMEM((4, CHUNK), jnp.int32),
            pltpu.VMEM((4, CHUNK), jnp.int32),
            pltpu.VMEM((CHUNK, D), jnp.float32),
            pltpu.VMEM((CHUNK, D), jnp.float32),
            pltpu.VMEM_SHARED((N_P, D), jnp.float32),
            pltpu.SemaphoreType.DMA,
            pltpu.SemaphoreType.DMA,
            pltpu.SemaphoreType.DMA,
            pltpu.SemaphoreType.DMA,
        ],
    )


def _dinv_bcast(deg0, deg1):
    deg = deg0 + deg1 + 1.0
    dinv = lax.rsqrt(deg)
    a = jnp.broadcast_to(dinv[None, :], (D, dinv.shape[0]))
    b = jnp.full((D, D), 1.0 / D, jnp.float32)
    return lax.dot_general(a, b, (((0,), (0,)), ((), ())),
                           preferred_element_type=jnp.float32)


def _tc1a_body(x_ref, w_ref, h_ref):
    h_ref[...] = jnp.dot(x_ref[...], w_ref[...],
                         preferred_element_type=jnp.float32)


def _tc1b_body(deg0_ref, deg1_ref, h_ref, m_ref, dinv_ref):
    dinvb = _dinv_bcast(deg0_ref[0, 0], deg1_ref[0, 0])
    dinv_ref[...] = dinvb
    m_ref[...] = dinvb * h_ref[...]


def _tc2_body(p0_ref, p1_ref, m_ref, dinv_ref, b_ref, w_ref, out_ref):
    s = p0_ref[0] + p1_ref[0] + m_ref[...]
    a = jnp.maximum(dinv_ref[...] * s + b_ref[...], 0.0)
    h = jnp.dot(a, w_ref[...], preferred_element_type=jnp.float32)
    out_ref[...] = dinv_ref[...] * h


def _tc3_body(p0_ref, p1_ref, m_ref, dinv_ref, b_ref, w_ref, bfc_ref, out_ref):
    s = p0_ref[0] + p1_ref[0] + m_ref[...]
    a = jnp.maximum(dinv_ref[...] * s + b_ref[...], 0.0)
    out_ref[...] = jnp.dot(a, w_ref[...],
                           preferred_element_type=jnp.float32) + bfc_ref[...]


_row_spec = pl.BlockSpec((ROW_BLK, D), lambda i: (i, 0))
_p0_spec = pl.BlockSpec((1, ROW_BLK, D), lambda i: (0, i, 0))
_p1_spec = pl.BlockSpec((1, ROW_BLK, D), lambda i: (1, i, 0))
_deg0_spec = pl.BlockSpec((1, 1, ROW_BLK), lambda i: (0, 0, i))
_deg1_spec = pl.BlockSpec((1, 1, ROW_BLK), lambda i: (1, 0, i))
_w_spec = pl.BlockSpec((D, D), lambda i: (0, 0))
_b_spec = pl.BlockSpec((1, D), lambda i: (0, 0))

_tc1a_call = pl.pallas_call(
    _tc1a_body,
    grid=(GRID,),
    in_specs=[_row_spec, _w_spec],
    out_specs=_row_spec,
    out_shape=jax.ShapeDtypeStruct((N_P, D), jnp.float32),
)

_tc1b_call = pl.pallas_call(
    _tc1b_body,
    grid=(GRID,),
    in_specs=[_deg0_spec, _deg1_spec, _row_spec],
    out_specs=[_row_spec, _row_spec],
    out_shape=[jax.ShapeDtypeStruct((N_P, D), jnp.float32),
               jax.ShapeDtypeStruct((N_P, D), jnp.float32)],
)

_tc2_call = pl.pallas_call(
    _tc2_body,
    grid=(GRID,),
    in_specs=[_p0_spec, _p1_spec, _row_spec, _row_spec, _b_spec, _w_spec],
    out_specs=_row_spec,
    out_shape=jax.ShapeDtypeStruct((N_P, D), jnp.float32),
)

_tc3_call = pl.pallas_call(
    _tc3_body,
    grid=(GRID,),
    in_specs=[_p0_spec, _p1_spec, _row_spec, _row_spec, _b_spec, _w_spec,
              _b_spec],
    out_specs=pl.BlockSpec((ROW_BLK, D), lambda i: (i, 0)),
    out_shape=jax.ShapeDtypeStruct((N_NODES, D), jnp.float32),
)


def kernel(x, edge_index, W1, b1, W2, b2, Wfc, bfc):
    e = jnp.asarray(edge_index, jnp.int32)
    n_pad = EDGES_P - e.shape[1]
    k = jnp.arange(n_pad, dtype=jnp.int32)
    src_p = jnp.concatenate([e[0], k % N_NODES]).reshape(NW, CHUNKS, CHUNK)
    dst_p = jnp.concatenate([e[1], N_NODES + k % (N_P - N_NODES)]
                            ).reshape(NW, CHUNKS, CHUNK)
    packed = (src_p << _DST_BITS) | dst_p

    zeros1 = jnp.zeros((N_P,), jnp.float32)
    zeros2 = jnp.zeros((N_P, D), jnp.float32)
    b1r = b1.reshape(1, D)
    b2r = b2.reshape(1, D)
    bfcr = bfc.reshape(1, D)

    deg = _deg_call()(dst_p, zeros1).reshape(NC, 1, N_P)
    h1 = _tc1a_call(x, W1)
    m1, dinvb = _tc1b_call(deg, deg, h1)
    p1 = _agg_call()(m1, packed, zeros2)
    m2 = _tc2_call(p1, p1, m1, dinvb, b1r, W2)
    p2 = _agg_call()(m2, packed, zeros2)
    return _tc3_call(p2, p2, m2, dinvb, b2r, Wfc, bfcr)


# speedup vs baseline: 1.1872x; 1.1872x over previous
import functools

import jax
import jax.numpy as jnp
from jax import lax
from jax.experimental import pallas as pl
from jax.experimental.pallas import tpu as pltpu
from jax.experimental.pallas import tpu_sc as plsc

N_NODES = 10000
D = 128
N_P = 10240
NC, NS = 2, 16
NW = NC * NS
ROWS_PER_TILE = N_P // NS
CHUNK = 64
CHUNKS = 160
EDGES_P = NW * CHUNKS * CHUNK
ROW_BLK = 1024
GRID = N_P // ROW_BLK

def _sc_mesh():
    return plsc.VectorSubcoreMesh(
        core_axis_name="c", subcore_axis_name="s", num_cores=NC, num_subcores=NS)


DCHUNK = 128
DCHUNKS = 80


def _deg_body(dst_hbm, out_hbm, didx, ones_v, zv, hist):
    cid = lax.axis_index("c")
    sid = lax.axis_index("s")
    wid = sid * NC + cid
    row0 = sid * ROWS_PER_TILE
    for i in range(ROWS_PER_TILE // 16):
        zv[pl.ds(i * 16, 16)] = jnp.zeros((16,), jnp.float32)
    pltpu.sync_copy(zv, hist.at[pl.ds(row0, ROWS_PER_TILE)])
    pltpu.sync_copy(dst_hbm.at[wid], didx)
    for i in range(DCHUNK // 16):
        ones_v[pl.ds(i * 16, 16)] = jnp.ones((16,), jnp.float32)
    plsc.subcore_barrier()

    def body(j, carry):
        pltpu.sync_copy(ones_v, hist.at[didx.at[j]], add=True)
        return carry

    lax.fori_loop(0, DCHUNKS, body, 0)
    plsc.subcore_barrier()
    pltpu.sync_copy(hist.at[pl.ds(row0, ROWS_PER_TILE)],
                    out_hbm.at[cid].at[0].at[pl.ds(row0, ROWS_PER_TILE)])


@functools.cache
def _deg_call():
    return pl.kernel(
        _deg_body,
        out_type=jax.ShapeDtypeStruct((NC, 1, N_P), jnp.float32),
        mesh=_sc_mesh(),
        scratch_types=[
            pltpu.VMEM((DCHUNKS, DCHUNK), jnp.int32),
            pltpu.VMEM((DCHUNK,), jnp.float32),
            pltpu.VMEM((ROWS_PER_TILE,), jnp.float32),
            pltpu.VMEM_SHARED((N_P,), jnp.float32),
        ],
    )


_DST_BITS = 14
_DST_MASK = (1 << _DST_BITS) - 1


def _agg_body(m_hbm, packed_hbm, out_hbm,
              pv, srcb, dstb, gbuf0, gbuf1, gbuf2, acc,
              gsem0, gsem1, gsem2, ssem0, ssem1, ssem2):
    cid = lax.axis_index("c")
    sid = lax.axis_index("s")
    wid = sid * NC + cid
    row0 = sid * ROWS_PER_TILE
    pltpu.sync_copy(packed_hbm.at[wid], pv)
    zrows = gbuf0.shape[0]

    def zbody(i, carry):
        gbuf0[i // 8, pl.ds((i % 8) * 16, 16)] = jnp.zeros((16,), jnp.float32)
        return carry

    lax.fori_loop(0, zrows * 8, zbody, 0)
    for z in range(ROWS_PER_TILE // zrows):
        pltpu.sync_copy(gbuf0, acc.at[pl.ds(row0 + z * zrows, zrows)])

    gbufs = (gbuf0, gbuf1, gbuf2)
    gsems = (gsem0, gsem1, gsem2)
    ssems = (ssem0, ssem1, ssem2)

    def unpack(j, slot):
        for k in range(CHUNK // 16):
            w = pv[j, pl.ds(k * 16, 16)]
            srcb[slot, pl.ds(k * 16, 16)] = lax.shift_right_logical(
                w, _DST_BITS)
            dstb[slot, pl.ds(k * 16, 16)] = lax.bitwise_and(w, _DST_MASK)

    def unpack_fire_gather(j, r3, r4):
        jw = jnp.where(j < CHUNKS, j, j - CHUNKS)
        unpack(jw, r4)
        pltpu.async_copy(m_hbm.at[srcb.at[r4]], gbufs[r3], gsems[r3])

    def wait_gather(r):
        pltpu.make_async_copy(m_hbm.at[srcb.at[0]], gbufs[r], gsems[r]).wait()

    def wait_scatter(r):
        pltpu.make_async_copy(gbufs[r], acc.at[dstb.at[0]], ssems[r]).wait()

    def step(j, r3, r4, first):
        wait_gather(r3)
        pltpu.async_copy(gbufs[r3], acc.at[dstb.at[r4]], ssems[r3], add=True)
        if not first:
            wait_scatter((r3 + 2) % 3)
        unpack_fire_gather(j + 2, (r3 + 2) % 3, (r4 + 2) % 4)

    plsc.subcore_barrier()
    unpack_fire_gather(0, 0, 0)
    unpack_fire_gather(1, 1, 1)
    for t in range(4):
        step(t, t % 3, t % 4, t == 0)

    def body(jj, carry):
        j0 = 12 * jj + 4
        for t in range(12):
            step(j0 + t, (4 + t) % 3, t % 4, False)
        return carry

    lax.fori_loop(0, (CHUNKS - 4) // 12, body, 0)
    wait_gather(CHUNKS % 3)
    wait_gather((CHUNKS + 1) % 3)
    wait_scatter((CHUNKS - 1) % 3)
    plsc.subcore_barrier()
    pltpu.sync_copy(acc.at[pl.ds(row0, ROWS_PER_TILE)],
                    out_hbm.at[cid].at[pl.ds(row0, ROWS_PER_TILE)])


@functools.cache
def _agg_call():
    return pl.kernel(
        _agg_body,
        out_type=jax.ShapeDtypeStruct((NC, N_P, D), jnp.float32),
        mesh=_sc_mesh(),
        scratch_types=[
            pltpu.VMEM((CHUNKS, CHUNK), jnp.int32),
            pltpu.VMEM((4, CHUNK), jnp.int32),
            pltpu.VMEM((4, CHUNK), jnp.int32),
            pltpu.VMEM((CHUNK, D), jnp.float32),
            pltpu.VMEM((CHUNK, D), jnp.float32),
            pltpu.VMEM((CHUNK, D), jnp.float32),
            pltpu.VMEM_SHARED((N_P, D), jnp.float32),
            pltpu.SemaphoreType.DMA,
            pltpu.SemaphoreType.DMA,
            pltpu.SemaphoreType.DMA,
            pltpu.SemaphoreType.DMA,
            pltpu.SemaphoreType.DMA,
            pltpu.SemaphoreType.DMA,
        ],
    )


def _dinv_bcast(deg0, deg1):
    deg = deg0 + deg1 + 1.0
    dinv = lax.rsqrt(deg)
    a = jnp.broadcast_to(dinv[None, :], (D, dinv.shape[0]))
    b = jnp.full((D, D), 1.0 / D, jnp.float32)
    return lax.dot_general(a, b, (((0,), (0,)), ((), ())),
                           preferred_element_type=jnp.float32)


def _tc1a_body(x_ref, w_ref, h_ref):
    h_ref[...] = jnp.dot(x_ref[...], w_ref[...],
                         preferred_element_type=jnp.float32)


def _tc1b_body(deg0_ref, deg1_ref, h_ref, m_ref, dinv_ref):
    dinvb = _dinv_bcast(deg0_ref[0, 0], deg1_ref[0, 0])
    dinv_ref[...] = dinvb
    m_ref[...] = dinvb * h_ref[...]


def _tc2_body(p0_ref, p1_ref, m_ref, dinv_ref, b_ref, w_ref, out_ref):
    s = p0_ref[0] + p1_ref[0] + m_ref[...]
    a = jnp.maximum(dinv_ref[...] * s + b_ref[...], 0.0)
    h = jnp.dot(a, w_ref[...], preferred_element_type=jnp.float32)
    out_ref[...] = dinv_ref[...] * h


def _tc3_body(p0_ref, p1_ref, m_ref, dinv_ref, b_ref, w_ref, bfc_ref, out_ref):
    s = p0_ref[0] + p1_ref[0] + m_ref[...]
    a = jnp.maximum(dinv_ref[...] * s + b_ref[...], 0.0)
    out_ref[...] = jnp.dot(a, w_ref[...],
                           preferred_element_type=jnp.float32) + bfc_ref[...]


_row_spec = pl.BlockSpec((ROW_BLK, D), lambda i: (i, 0))
_p0_spec = pl.BlockSpec((1, ROW_BLK, D), lambda i: (0, i, 0))
_p1_spec = pl.BlockSpec((1, ROW_BLK, D), lambda i: (1, i, 0))
_deg0_spec = pl.BlockSpec((1, 1, ROW_BLK), lambda i: (0, 0, i))
_deg1_spec = pl.BlockSpec((1, 1, ROW_BLK), lambda i: (1, 0, i))
_w_spec = pl.BlockSpec((D, D), lambda i: (0, 0))
_b_spec = pl.BlockSpec((1, D), lambda i: (0, 0))

_tc1a_call = pl.pallas_call(
    _tc1a_body,
    grid=(GRID,),
    in_specs=[_row_spec, _w_spec],
    out_specs=_row_spec,
    out_shape=jax.ShapeDtypeStruct((N_P, D), jnp.float32),
)

_tc1b_call = pl.pallas_call(
    _tc1b_body,
    grid=(GRID,),
    in_specs=[_deg0_spec, _deg1_spec, _row_spec],
    out_specs=[_row_spec, _row_spec],
    out_shape=[jax.ShapeDtypeStruct((N_P, D), jnp.float32),
               jax.ShapeDtypeStruct((N_P, D), jnp.float32)],
)

_tc2_call = pl.pallas_call(
    _tc2_body,
    grid=(GRID,),
    in_specs=[_p0_spec, _p1_spec, _row_spec, _row_spec, _b_spec, _w_spec],
    out_specs=_row_spec,
    out_shape=jax.ShapeDtypeStruct((N_P, D), jnp.float32),
)

_tc3_call = pl.pallas_call(
    _tc3_body,
    grid=(GRID,),
    in_specs=[_p0_spec, _p1_spec, _row_spec, _row_spec, _b_spec, _w_spec,
              _b_spec],
    out_specs=pl.BlockSpec((ROW_BLK, D), lambda i: (i, 0)),
    out_shape=jax.ShapeDtypeStruct((N_NODES, D), jnp.float32),
)


import numpy as _np

_N_PAD = EDGES_P - 320000
_PAD_SRC = _np.arange(_N_PAD, dtype=_np.int32) % N_NODES
_PAD_DST = N_NODES + _np.arange(_N_PAD, dtype=_np.int32) % (N_P - N_NODES)
_PAD_PACKED = (_PAD_SRC << _DST_BITS) | _PAD_DST


def kernel(x, edge_index, W1, b1, W2, b2, Wfc, bfc):
    e = jnp.asarray(edge_index, jnp.int32)
    packed = jnp.concatenate(
        [(e[0] << _DST_BITS) | e[1], jnp.asarray(_PAD_PACKED)]
    ).reshape(NW, CHUNKS, CHUNK)
    dst_p = jnp.concatenate([e[1], jnp.asarray(_PAD_DST)]
                            ).reshape(NW, DCHUNKS, DCHUNK)

    b1r = b1.reshape(1, D)
    b2r = b2.reshape(1, D)
    bfcr = bfc.reshape(1, D)

    deg = _deg_call()(dst_p)
    h1 = _tc1a_call(x, W1)
    m1, dinvb = _tc1b_call(deg, deg, h1)
    p1 = _agg_call()(m1, packed)
    m2 = _tc2_call(p1, p1, m1, dinvb, b1r, W2)
    p2 = _agg_call()(m2, packed)
    return _tc3_call(p2, p2, m2, dinvb, b2r, Wfc, bfcr)


# speedup vs baseline: 1.1988x; 1.0098x over previous
import functools

import jax
import jax.numpy as jnp
from jax import lax
from jax.experimental import pallas as pl
from jax.experimental.pallas import tpu as pltpu
from jax.experimental.pallas import tpu_sc as plsc

N_NODES = 10000
D = 128
N_P = 10240
NC, NS = 2, 16
NW = NC * NS
ROWS_PER_TILE = N_P // NS
CHUNK = 64
CHUNKS = 160
EDGES_P = NW * CHUNKS * CHUNK
ROW_BLK = 1024
GRID = N_P // ROW_BLK

def _sc_mesh():
    return plsc.VectorSubcoreMesh(
        core_axis_name="c", subcore_axis_name="s", num_cores=NC, num_subcores=NS)


DCHUNK = 128
DCHUNKS = 80


def _deg_body(dst_hbm, out_hbm, didx, ones_v, zv, hist):
    cid = lax.axis_index("c")
    sid = lax.axis_index("s")
    wid = sid * NC + cid
    row0 = sid * ROWS_PER_TILE
    for i in range(ROWS_PER_TILE // 16):
        zv[pl.ds(i * 16, 16)] = jnp.zeros((16,), jnp.float32)
    pltpu.sync_copy(zv, hist.at[pl.ds(row0, ROWS_PER_TILE)])
    pltpu.sync_copy(dst_hbm.at[wid], didx)
    for i in range(DCHUNK // 16):
        ones_v[pl.ds(i * 16, 16)] = jnp.ones((16,), jnp.float32)
    plsc.subcore_barrier()

    def body(j, carry):
        pltpu.sync_copy(ones_v, hist.at[didx.at[j]], add=True)
        return carry

    lax.fori_loop(0, DCHUNKS, body, 0)
    plsc.subcore_barrier()
    pltpu.sync_copy(hist.at[pl.ds(row0, ROWS_PER_TILE)],
                    out_hbm.at[cid].at[0].at[pl.ds(row0, ROWS_PER_TILE)])


@functools.cache
def _deg_call():
    return pl.kernel(
        _deg_body,
        out_type=jax.ShapeDtypeStruct((NC, 1, N_P), jnp.float32),
        mesh=_sc_mesh(),
        scratch_types=[
            pltpu.VMEM((DCHUNKS, DCHUNK), jnp.int32),
            pltpu.VMEM((DCHUNK,), jnp.float32),
            pltpu.VMEM((ROWS_PER_TILE,), jnp.float32),
            pltpu.VMEM_SHARED((N_P,), jnp.float32),
        ],
    )


_DST_BITS = 14
_DST_MASK = (1 << _DST_BITS) - 1


def _agg_body(m_hbm, packed_hbm, out_hbm,
              pv, srcb, dstb, gbuf0, gbuf1, gbuf2, acc,
              gsem0, gsem1, gsem2, ssem0, ssem1, ssem2):
    cid = lax.axis_index("c")
    sid = lax.axis_index("s")
    wid = sid * NC + cid
    row0 = sid * ROWS_PER_TILE
    pltpu.sync_copy(packed_hbm.at[wid], pv)
    zrows = gbuf0.shape[0]

    def zbody(i, carry):
        gbuf0[i // 8, pl.ds((i % 8) * 16, 16)] = jnp.zeros((16,), jnp.float32)
        return carry

    lax.fori_loop(0, zrows * 8, zbody, 0)
    for z in range(ROWS_PER_TILE // zrows):
        pltpu.sync_copy(gbuf0, acc.at[pl.ds(row0 + z * zrows, zrows)])

    gbufs = (gbuf0, gbuf1, gbuf2)
    gsems = (gsem0, gsem1, gsem2)
    ssems = (ssem0, ssem1, ssem2)

    def unpack(j, slot):
        for k in range(CHUNK // 16):
            w = pv[j, pl.ds(k * 16, 16)]
            srcb[slot, pl.ds(k * 16, 16)] = lax.shift_right_logical(
                w, _DST_BITS)
            dstb[slot, pl.ds(k * 16, 16)] = lax.bitwise_and(w, _DST_MASK)

    def unpack_fire_gather(j, r3, r4):
        jw = jnp.where(j < CHUNKS, j, j - CHUNKS)
        unpack(jw, r4)
        pltpu.async_copy(m_hbm.at[srcb.at[r4]], gbufs[r3], gsems[r3])

    def wait_gather(r):
        pltpu.make_async_copy(m_hbm.at[srcb.at[0]], gbufs[r], gsems[r]).wait()

    def wait_scatter(r):
        pltpu.make_async_copy(gbufs[r], acc.at[dstb.at[0]], ssems[r]).wait()

    def step(j, r3, r4, first):
        wait_gather(r3)
        pltpu.async_copy(gbufs[r3], acc.at[dstb.at[r4]], ssems[r3], add=True)
        if not first:
            wait_scatter((r3 + 2) % 3)
        unpack_fire_gather(j + 2, (r3 + 2) % 3, (r4 + 2) % 4)

    plsc.subcore_barrier()
    unpack_fire_gather(0, 0, 0)
    unpack_fire_gather(1, 1, 1)
    for t in range(4):
        step(t, t % 3, t % 4, t == 0)

    def body(jj, carry):
        j0 = 12 * jj + 4
        for t in range(12):
            step(j0 + t, (4 + t) % 3, t % 4, False)
        return carry

    lax.fori_loop(0, (CHUNKS - 4) // 12, body, 0)
    wait_gather(CHUNKS % 3)
    wait_gather((CHUNKS + 1) % 3)
    wait_scatter((CHUNKS - 1) % 3)
    plsc.subcore_barrier()
    pltpu.sync_copy(acc.at[pl.ds(row0, ROWS_PER_TILE)],
                    out_hbm.at[cid].at[pl.ds(row0, ROWS_PER_TILE)])


@functools.cache
def _agg_call():
    return pl.kernel(
        _agg_body,
        out_type=jax.ShapeDtypeStruct((NC, N_P, D), jnp.float32),
        mesh=_sc_mesh(),
        scratch_types=[
            pltpu.VMEM((CHUNKS, CHUNK), jnp.int32),
            pltpu.VMEM((4, CHUNK), jnp.int32),
            pltpu.VMEM((4, CHUNK), jnp.int32),
            pltpu.VMEM((CHUNK, D), jnp.float32),
            pltpu.VMEM((CHUNK, D), jnp.float32),
            pltpu.VMEM((CHUNK, D), jnp.float32),
            pltpu.VMEM_SHARED((N_P, D), jnp.float32),
            pltpu.SemaphoreType.DMA,
            pltpu.SemaphoreType.DMA,
            pltpu.SemaphoreType.DMA,
            pltpu.SemaphoreType.DMA,
            pltpu.SemaphoreType.DMA,
            pltpu.SemaphoreType.DMA,
        ],
    )


def _dinv_bcast(deg0, deg1):
    deg = deg0 + deg1 + 1.0
    dinv = lax.rsqrt(deg)
    a = jnp.broadcast_to(dinv[None, :], (D, dinv.shape[0]))
    b = jnp.full((D, D), 1.0 / D, jnp.float32)
    return lax.dot_general(a, b, (((0,), (0,)), ((), ())),
                           preferred_element_type=jnp.float32)


_E_BLK = EDGES_P // 10


def _pack_body(e_ref, packed_ref, dst_ref):
    i = pl.program_id(0)
    g = lax.iota(jnp.int32, _E_BLK) + i * _E_BLK
    real = g < 320000
    pad_dst = N_NODES + g % (N_P - N_NODES)
    pad_src = g % N_NODES
    src = jnp.where(real, e_ref[0], pad_src)
    dst = jnp.where(real, e_ref[1], pad_dst)
    packed_ref[...] = (src << _DST_BITS) | dst
    dst_ref[...] = dst


def _tc1a_body(x_ref, w_ref, h_ref):
    h_ref[...] = jnp.dot(x_ref[...], w_ref[...],
                         preferred_element_type=jnp.float32)


def _tc1b_body(deg0_ref, deg1_ref, h_ref, m_ref):
    dinvb = _dinv_bcast(deg0_ref[0, 0], deg1_ref[0, 0])
    m_ref[...] = dinvb * h_ref[...]


def _tc2_body(p0_ref, p1_ref, m_ref, deg0_ref, deg1_ref, b_ref, w_ref,
              out_ref):
    dinvb = _dinv_bcast(deg0_ref[0, 0], deg1_ref[0, 0])
    s = p0_ref[0] + p1_ref[0] + m_ref[...]
    a = jnp.maximum(dinvb * s + b_ref[...], 0.0)
    h = jnp.dot(a, w_ref[...], preferred_element_type=jnp.float32)
    out_ref[...] = dinvb * h


def _tc3_body(p0_ref, p1_ref, m_ref, deg0_ref, deg1_ref, b_ref, w_ref,
              bfc_ref, out_ref):
    dinvb = _dinv_bcast(deg0_ref[0, 0], deg1_ref[0, 0])
    s = p0_ref[0] + p1_ref[0] + m_ref[...]
    a = jnp.maximum(dinvb * s + b_ref[...], 0.0)
    out_ref[...] = jnp.dot(a, w_ref[...],
                           preferred_element_type=jnp.float32) + bfc_ref[...]


_row_spec = pl.BlockSpec((ROW_BLK, D), lambda i: (i, 0))
_p0_spec = pl.BlockSpec((1, ROW_BLK, D), lambda i: (0, i, 0))
_p1_spec = pl.BlockSpec((1, ROW_BLK, D), lambda i: (1, i, 0))
_deg0_spec = pl.BlockSpec((1, 1, ROW_BLK), lambda i: (0, 0, i))
_deg1_spec = pl.BlockSpec((1, 1, ROW_BLK), lambda i: (1, 0, i))
_w_spec = pl.BlockSpec((D, D), lambda i: (0, 0))
_b_spec = pl.BlockSpec((1, D), lambda i: (0, 0))

_pack_call = pl.pallas_call(
    _pack_body,
    grid=(10,),
    in_specs=[pl.BlockSpec((2, _E_BLK), lambda i: (0, i))],
    out_specs=[pl.BlockSpec((_E_BLK,), lambda i: (i,)),
               pl.BlockSpec((_E_BLK,), lambda i: (i,))],
    out_shape=[jax.ShapeDtypeStruct((EDGES_P,), jnp.int32),
               jax.ShapeDtypeStruct((EDGES_P,), jnp.int32)],
)

_tc1a_call = pl.pallas_call(
    _tc1a_body,
    grid=(GRID,),
    in_specs=[_row_spec, _w_spec],
    out_specs=_row_spec,
    out_shape=jax.ShapeDtypeStruct((N_P, D), jnp.float32),
)

_tc1b_call = pl.pallas_call(
    _tc1b_body,
    grid=(GRID,),
    in_specs=[_deg0_spec, _deg1_spec, _row_spec],
    out_specs=_row_spec,
    out_shape=jax.ShapeDtypeStruct((N_P, D), jnp.float32),
)

_tc2_call = pl.pallas_call(
    _tc2_body,
    grid=(GRID,),
    in_specs=[_p0_spec, _p1_spec, _row_spec, _deg0_spec, _deg1_spec, _b_spec,
              _w_spec],
    out_specs=_row_spec,
    out_shape=jax.ShapeDtypeStruct((N_P, D), jnp.float32),
)

_tc3_call = pl.pallas_call(
    _tc3_body,
    grid=(GRID,),
    in_specs=[_p0_spec, _p1_spec, _row_spec, _deg0_spec, _deg1_spec, _b_spec,
              _w_spec, _b_spec],
    out_specs=pl.BlockSpec((ROW_BLK, D), lambda i: (i, 0)),
    out_shape=jax.ShapeDtypeStruct((N_NODES, D), jnp.float32),
)


def kernel(x, edge_index, W1, b1, W2, b2, Wfc, bfc):
    e = jnp.asarray(edge_index, jnp.int32)
    packed, dst_p = _pack_call(e)
    packed = packed.reshape(NW, CHUNKS, CHUNK)
    dst_p = dst_p.reshape(NW, DCHUNKS, DCHUNK)

    b1r = b1.reshape(1, D)
    b2r = b2.reshape(1, D)
    bfcr = bfc.reshape(1, D)

    deg = _deg_call()(dst_p)
    h1 = _tc1a_call(x, W1)
    m1 = _tc1b_call(deg, deg, h1)
    p1 = _agg_call()(m1, packed)
    m2 = _tc2_call(p1, p1, m1, deg, deg, b1r, W2)
    p2 = _agg_call()(m2, packed)
    return _tc3_call(p2, p2, m2, deg, deg, b2r, Wfc, bfcr)


# speedup vs baseline: 1.2068x; 1.0066x over previous
import functools

import jax
import jax.numpy as jnp
from jax import lax
from jax.experimental import pallas as pl
from jax.experimental.pallas import tpu as pltpu
from jax.experimental.pallas import tpu_sc as plsc

N_NODES = 10000
D = 128
N_P = 10240
NC, NS = 2, 16
NW = NC * NS
ROWS_PER_TILE = N_P // NS
CHUNK = 64
CHUNKS = 160
EDGES_P = NW * CHUNKS * CHUNK
ROW_BLK = 1024
GRID = N_P // ROW_BLK

def _sc_mesh():
    return plsc.VectorSubcoreMesh(
        core_axis_name="c", subcore_axis_name="s", num_cores=NC, num_subcores=NS)


DCHUNK = 128
DCHUNKS = 80


def _deg_body(dst_hbm, out_hbm, didx, ones_v, zv, hist):
    cid = lax.axis_index("c")
    sid = lax.axis_index("s")
    wid = sid * NC + cid
    row0 = sid * ROWS_PER_TILE
    for i in range(ROWS_PER_TILE // 16):
        zv[pl.ds(i * 16, 16)] = jnp.zeros((16,), jnp.float32)
    pltpu.sync_copy(zv, hist.at[pl.ds(row0, ROWS_PER_TILE)])
    pltpu.sync_copy(dst_hbm.at[wid], didx)
    for i in range(DCHUNK // 16):
        ones_v[pl.ds(i * 16, 16)] = jnp.ones((16,), jnp.float32)
    plsc.subcore_barrier()

    def body(j, carry):
        pltpu.sync_copy(ones_v, hist.at[didx.at[j]], add=True)
        return carry

    lax.fori_loop(0, DCHUNKS, body, 0)
    plsc.subcore_barrier()
    pltpu.sync_copy(hist.at[pl.ds(row0, ROWS_PER_TILE)],
                    out_hbm.at[cid].at[0].at[pl.ds(row0, ROWS_PER_TILE)])


@functools.cache
def _deg_call():
    return pl.kernel(
        _deg_body,
        out_type=jax.ShapeDtypeStruct((NC, 1, N_P), jnp.float32),
        mesh=_sc_mesh(),
        scratch_types=[
            pltpu.VMEM((DCHUNKS, DCHUNK), jnp.int32),
            pltpu.VMEM((DCHUNK,), jnp.float32),
            pltpu.VMEM((ROWS_PER_TILE,), jnp.float32),
            pltpu.VMEM_SHARED((N_P,), jnp.float32),
        ],
    )


_DST_BITS = 14
_DST_MASK = (1 << _DST_BITS) - 1


def _agg_body(m_hbm, packed_hbm, out_hbm,
              pv, srcb, dstb, gbuf0, gbuf1, gbuf2, acc,
              gsem0, gsem1, gsem2, ssem0, ssem1, ssem2):
    cid = lax.axis_index("c")
    sid = lax.axis_index("s")
    wid = sid * NC + cid
    row0 = sid * ROWS_PER_TILE
    pltpu.sync_copy(packed_hbm.at[wid], pv)
    zrows = gbuf0.shape[0]

    def zbody(i, carry):
        gbuf0[i // 8, pl.ds((i % 8) * 16, 16)] = jnp.zeros((16,), jnp.float32)
        return carry

    lax.fori_loop(0, zrows * 8, zbody, 0)
    for z in range(ROWS_PER_TILE // zrows):
        pltpu.sync_copy(gbuf0, acc.at[pl.ds(row0 + z * zrows, zrows)])

    gbufs = (gbuf0, gbuf1, gbuf2)
    gsems = (gsem0, gsem1, gsem2)
    ssems = (ssem0, ssem1, ssem2)

    def unpack(j, slot):
        for k in range(CHUNK // 16):
            w = pv[j, pl.ds(k * 16, 16)]
            srcb[slot, pl.ds(k * 16, 16)] = lax.shift_right_logical(
                w, _DST_BITS)
            dstb[slot, pl.ds(k * 16, 16)] = lax.bitwise_and(w, _DST_MASK)

    def unpack_fire_gather(j, r3, r4):
        jw = jnp.where(j < CHUNKS, j, j - CHUNKS)
        unpack(jw, r4)
        pltpu.async_copy(m_hbm.at[srcb.at[r4]], gbufs[r3], gsems[r3])

    def wait_gather(r):
        pltpu.make_async_copy(m_hbm.at[srcb.at[0]], gbufs[r], gsems[r]).wait()

    def wait_scatter(r):
        pltpu.make_async_copy(gbufs[r], acc.at[dstb.at[0]], ssems[r]).wait()

    def step(j, r3, r4, first):
        wait_gather(r3)
        pltpu.async_copy(gbufs[r3], acc.at[dstb.at[r4]], ssems[r3], add=True)
        if not first:
            wait_scatter((r3 + 2) % 3)
        unpack_fire_gather(j + 2, (r3 + 2) % 3, (r4 + 2) % 4)

    plsc.subcore_barrier()
    unpack_fire_gather(0, 0, 0)
    unpack_fire_gather(1, 1, 1)
    for t in range(4):
        step(t, t % 3, t % 4, t == 0)

    def body(jj, carry):
        j0 = 12 * jj + 4
        for t in range(12):
            step(j0 + t, (4 + t) % 3, t % 4, False)
        return carry

    lax.fori_loop(0, (CHUNKS - 4) // 12, body, 0)
    wait_gather(CHUNKS % 3)
    wait_gather((CHUNKS + 1) % 3)
    wait_scatter((CHUNKS - 1) % 3)
    plsc.subcore_barrier()
    pltpu.sync_copy(acc.at[pl.ds(row0, ROWS_PER_TILE)],
                    out_hbm.at[cid].at[pl.ds(row0, ROWS_PER_TILE)])


@functools.cache
def _agg_call():
    return pl.kernel(
        _agg_body,
        out_type=jax.ShapeDtypeStruct((NC, N_P, D), jnp.float32),
        mesh=_sc_mesh(),
        scratch_types=[
            pltpu.VMEM((CHUNKS, CHUNK), jnp.int32),
            pltpu.VMEM((4, CHUNK), jnp.int32),
            pltpu.VMEM((4, CHUNK), jnp.int32),
            pltpu.VMEM((CHUNK, D), jnp.float32),
            pltpu.VMEM((CHUNK, D), jnp.float32),
            pltpu.VMEM((CHUNK, D), jnp.float32),
            pltpu.VMEM_SHARED((N_P, D), jnp.float32),
            pltpu.SemaphoreType.DMA,
            pltpu.SemaphoreType.DMA,
            pltpu.SemaphoreType.DMA,
            pltpu.SemaphoreType.DMA,
            pltpu.SemaphoreType.DMA,
            pltpu.SemaphoreType.DMA,
        ],
    )


def _dinv_bcast(deg0, deg1):
    deg = deg0 + deg1 + 1.0
    dinv = lax.rsqrt(deg)
    a = jnp.broadcast_to(dinv[None, :], (D, dinv.shape[0]))
    b = jnp.full((D, D), 1.0 / D, jnp.float32)
    return lax.dot_general(a, b, (((0,), (0,)), ((), ())),
                           preferred_element_type=jnp.float32)


_E_ROWS = EDGES_P // 128
_E_RBLK = _E_ROWS // 10
_E_REAL_ROWS = 320000 // 128


def _pack_body(e0_ref, e1_ref, packed_ref, dst_ref):
    i = pl.program_id(0)
    row = lax.broadcasted_iota(jnp.int32, (_E_RBLK, 128), 0) + i * _E_RBLK
    lane = lax.broadcasted_iota(jnp.int32, (_E_RBLK, 128), 1)
    g = row * 128 + lane
    real = row < _E_REAL_ROWS
    pad_dst = N_NODES + g % (N_P - N_NODES)
    pad_src = g % N_NODES
    src = jnp.where(real, e0_ref[0], pad_src)
    dst = jnp.where(real, e1_ref[0], pad_dst)
    packed_ref[...] = (src << _DST_BITS) | dst
    dst_ref[...] = dst


def _tc1a_body(x_ref, w_ref, h_ref):
    h_ref[...] = jnp.dot(x_ref[...], w_ref[...],
                         preferred_element_type=jnp.float32)


def _tc1b_body(deg0_ref, deg1_ref, h_ref, m_ref):
    dinvb = _dinv_bcast(deg0_ref[0, 0], deg1_ref[0, 0])
    m_ref[...] = dinvb * h_ref[...]


def _tc2_body(p0_ref, p1_ref, m_ref, deg0_ref, deg1_ref, b_ref, w_ref,
              out_ref):
    dinvb = _dinv_bcast(deg0_ref[0, 0], deg1_ref[0, 0])
    s = p0_ref[0] + p1_ref[0] + m_ref[...]
    a = jnp.maximum(dinvb * s + b_ref[...], 0.0)
    h = jnp.dot(a, w_ref[...], preferred_element_type=jnp.float32)
    out_ref[...] = dinvb * h


def _tc3_body(p0_ref, p1_ref, m_ref, deg0_ref, deg1_ref, b_ref, w_ref,
              bfc_ref, out_ref):
    dinvb = _dinv_bcast(deg0_ref[0, 0], deg1_ref[0, 0])
    s = p0_ref[0] + p1_ref[0] + m_ref[...]
    a = jnp.maximum(dinvb * s + b_ref[...], 0.0)
    out_ref[...] = jnp.dot(a, w_ref[...],
                           preferred_element_type=jnp.float32) + bfc_ref[...]


_row_spec = pl.BlockSpec((ROW_BLK, D), lambda i: (i, 0))
_p0_spec = pl.BlockSpec((1, ROW_BLK, D), lambda i: (0, i, 0))
_p1_spec = pl.BlockSpec((1, ROW_BLK, D), lambda i: (1, i, 0))
_deg0_spec = pl.BlockSpec((1, 1, ROW_BLK), lambda i: (0, 0, i))
_deg1_spec = pl.BlockSpec((1, 1, ROW_BLK), lambda i: (1, 0, i))
_w_spec = pl.BlockSpec((D, D), lambda i: (0, 0))
_b_spec = pl.BlockSpec((1, D), lambda i: (0, 0))

_pack_call = pl.pallas_call(
    _pack_body,
    grid=(10,),
    in_specs=[pl.BlockSpec((1, _E_RBLK, 128), lambda i: (0, i, 0)),
              pl.BlockSpec((1, _E_RBLK, 128), lambda i: (1, i, 0))],
    out_specs=[pl.BlockSpec((_E_RBLK, 128), lambda i: (i, 0)),
               pl.BlockSpec((_E_RBLK, 128), lambda i: (i, 0))],
    out_shape=[jax.ShapeDtypeStruct((_E_ROWS, 128), jnp.int32),
               jax.ShapeDtypeStruct((_E_ROWS, 128), jnp.int32)],
)

_tc1a_call = pl.pallas_call(
    _tc1a_body,
    grid=(GRID,),
    in_specs=[_row_spec, _w_spec],
    out_specs=_row_spec,
    out_shape=jax.ShapeDtypeStruct((N_P, D), jnp.float32),
)

_tc1b_call = pl.pallas_call(
    _tc1b_body,
    grid=(GRID,),
    in_specs=[_deg0_spec, _deg1_spec, _row_spec],
    out_specs=_row_spec,
    out_shape=jax.ShapeDtypeStruct((N_P, D), jnp.float32),
)

_tc2_call = pl.pallas_call(
    _tc2_body,
    grid=(GRID,),
    in_specs=[_p0_spec, _p1_spec, _row_spec, _deg0_spec, _deg1_spec, _b_spec,
              _w_spec],
    out_specs=_row_spec,
    out_shape=jax.ShapeDtypeStruct((N_P, D), jnp.float32),
)

_tc3_call = pl.pallas_call(
    _tc3_body,
    grid=(GRID,),
    in_specs=[_p0_spec, _p1_spec, _row_spec, _deg0_spec, _deg1_spec, _b_spec,
              _w_spec, _b_spec],
    out_specs=pl.BlockSpec((ROW_BLK, D), lambda i: (i, 0)),
    out_shape=jax.ShapeDtypeStruct((N_NODES, D), jnp.float32),
)


def kernel(x, edge_index, W1, b1, W2, b2, Wfc, bfc):
    e = jnp.asarray(edge_index, jnp.int32).reshape(2, _E_REAL_ROWS, 128)
    packed, dst_p = _pack_call(e, e)
    packed = packed.reshape(NW, CHUNKS, CHUNK)
    dst_p = dst_p.reshape(NW, DCHUNKS, DCHUNK)

    b1r = b1.reshape(1, D)
    b2r = b2.reshape(1, D)
    bfcr = bfc.reshape(1, D)

    deg = _deg_call()(dst_p)
    h1 = _tc1a_call(x, W1)
    m1 = _tc1b_call(deg, deg, h1)
    p1 = _agg_call()(m1, packed)
    m2 = _tc2_call(p1, p1, m1, deg, deg, b1r, W2)
    p2 = _agg_call()(m2, packed)
    return _tc3_call(p2, p2, m2, deg, deg, b2r, Wfc, bfcr)
